# baseline (speedup 1.0000x reference)
import functools

import jax
import jax.numpy as jnp
from jax.experimental import pallas as pl
from jax.experimental.pallas import tpu as pltpu
from jax.experimental.pallas import tpu_sc as plsc

_BATCH = 16384
_SEQ = 30
_EMB = 64
_HIDD = 50
_CLASS = 128
_N_IDX = _BATCH * _SEQ
_VHALF = 500000

_NW = 32
_B_PER_W = _N_IDX // _NW
_GCHUNK = 512
_N_GCH = _B_PER_W // _GCHUNK

_CHUNK = 1024
_W = 2 * _EMB
_H2 = 2 * _HIDD


def _sc_gather(table128, idx_half):
    mesh = plsc.VectorSubcoreMesh(core_axis_name="c", subcore_axis_name="s")

    @functools.partial(
        pl.kernel,
        out_type=jax.ShapeDtypeStruct((_N_IDX, _W), jnp.float32),
        mesh=mesh,
        scratch_types=[
            pltpu.VMEM((_GCHUNK,), jnp.int32),
            pltpu.VMEM((_GCHUNK, _W), jnp.float32),
            pltpu.SemaphoreType.DMA,
        ],
    )
    def gather_kernel(table_hbm, idx_hbm, out_hbm, idx_v, rows_v, sem):
        wid = jax.lax.axis_index("s") * 2 + jax.lax.axis_index("c")
        base = wid * _B_PER_W

        @pl.loop(0, _N_GCH)
        def _(j):
            off = base + j * _GCHUNK
            pltpu.sync_copy(idx_hbm.at[pl.ds(off, _GCHUNK)], idx_v)
            pltpu.async_copy(table_hbm.at[idx_v], rows_v, sem).wait()
            pltpu.sync_copy(rows_v, out_hbm.at[pl.ds(off, _GCHUNK)])

    return gather_kernel(table128, idx_half)


def _mlp_body(e_ref, m_ref, w1d_ref, b1d_ref, w2a_ref, b2a_ref, w2b_ref,
              b2b_ref, w2c_ref, b2c_ref, o_ref):
    w1d = w1d_ref[...]
    b1d = b1d_ref[...]
    rows = e_ref.shape[0]
    lane = jax.lax.broadcasted_iota(jnp.int32, (rows, _H2), 1)
    first_half = lane < _HIDD
    acc = jnp.zeros((rows, _CLASS), jnp.float32)
    for s in range(_SEQ):
        es = e_ref[:, s * _W:(s + 1) * _W].astype(jnp.bfloat16)
        h_both = jnp.tanh(
            jnp.dot(es, w1d, preferred_element_type=jnp.float32) + b1d)
        m = m_ref[:, s:s + 1]
        mask = jnp.where(first_half, 1.0 - m, m)
        acc = acc + jnp.dot((h_both * mask).astype(jnp.bfloat16), w2a_ref[s],
                            preferred_element_type=jnp.float32)
    h2 = jnp.tanh(acc + b2a_ref[...]).astype(jnp.bfloat16)
    h3 = jnp.tanh(
        jnp.dot(h2, w2b_ref[...], preferred_element_type=jnp.float32)
        + b2b_ref[...]).astype(jnp.bfloat16)
    o_ref[...] = (
        jnp.dot(h3, w2c_ref[...], preferred_element_type=jnp.float32)
        + b2c_ref[...])


def _tc_mlp(e2, m, w1d, b1d, w2ad, b2a, w2b, b2b, w2c, b2c):
    return pl.pallas_call(
        _mlp_body,
        grid=(_BATCH // _CHUNK,),
        in_specs=[
            pl.BlockSpec((_CHUNK, _SEQ * _W), lambda i: (i, 0)),
            pl.BlockSpec((_CHUNK, _SEQ), lambda i: (i, 0)),
            pl.BlockSpec((_W, _H2), lambda i: (0, 0)),
            pl.BlockSpec((1, _H2), lambda i: (0, 0)),
            pl.BlockSpec((_SEQ, _H2, _CLASS), lambda i: (0, 0, 0)),
            pl.BlockSpec((1, _CLASS), lambda i: (0, 0)),
            pl.BlockSpec((_CLASS, 64), lambda i: (0, 0)),
            pl.BlockSpec((1, 64), lambda i: (0, 0)),
            pl.BlockSpec((64, _CLASS), lambda i: (0, 0)),
            pl.BlockSpec((1, _CLASS), lambda i: (0, 0)),
        ],
        out_specs=pl.BlockSpec((_CHUNK, _CLASS), lambda i: (i, 0)),
        out_shape=jax.ShapeDtypeStruct((_BATCH, _CLASS), jnp.float32),
    )(e2, m, w1d, b1d, w2ad, b2a, w2b, b2b, w2c, b2c)


def kernel(x, table, W1, b1, W2a, b2a, W2b, b2b, W2c, b2c):
    table128 = table.reshape(_VHALF, _W)
    idx = x.reshape(_N_IDX)
    idx_half = jax.lax.shift_right_logical(idx, 1)
    parity = jnp.bitwise_and(x, 1).astype(jnp.float32)
    e = _sc_gather(table128, idx_half)
    e2 = e.reshape(_BATCH, _SEQ * _W)
    w1d = (jnp.zeros((_W, _H2), jnp.float32)
           .at[:_EMB, :_HIDD].set(W1)
           .at[_EMB:, _HIDD:].set(W1)).astype(jnp.bfloat16)
    b1d = jnp.concatenate([b1, b1]).reshape(1, _H2)
    w2a3 = W2a.reshape(_SEQ, _HIDD, _CLASS)
    w2ad = jnp.concatenate([w2a3, w2a3], axis=1).astype(jnp.bfloat16)
    return _tc_mlp(e2, parity, w1d, b1d, w2ad, b2a.reshape(1, -1),
                   W2b.astype(jnp.bfloat16), b2b.reshape(1, -1),
                   W2c.astype(jnp.bfloat16), b2c.reshape(1, -1))


# speedup vs baseline: 2.1991x; 2.1991x over previous
import functools

import jax
import jax.numpy as jnp
from jax.experimental import pallas as pl
from jax.experimental.pallas import tpu as pltpu
from jax.experimental.pallas import tpu_sc as plsc

_BATCH = 16384
_SEQ = 30
_EMB = 64
_HIDD = 50
_CLASS = 128
_NUM_EMB = 1000000
_N_IDX = _BATCH * _SEQ
_W = 128

_NW = 32
_B_PER_W = _N_IDX // _NW
_GCHUNK = 512
_N_GCH = _B_PER_W // _GCHUNK

_PCHUNK = 8192
_CHUNK = 512


def _proj_body(t_ref, w1p_ref, b1p_ref, o_ref):
    t = t_ref[...].astype(jnp.bfloat16)
    h = jax.lax.dot_general(
        t, w1p_ref[...], dimension_numbers=(((0,), (0,)), ((), ())),
        preferred_element_type=jnp.float32)
    o_ref[...] = jnp.tanh(h + b1p_ref[...])


def _tc_project(tableT, w1p, b1p):
    grid = (_NUM_EMB + _PCHUNK - 1) // _PCHUNK
    return pl.pallas_call(
        _proj_body,
        grid=(grid,),
        in_specs=[
            pl.BlockSpec((_EMB, _PCHUNK), lambda i: (0, i)),
            pl.BlockSpec((_EMB, _W), lambda i: (0, 0)),
            pl.BlockSpec((1, _W), lambda i: (0, 0)),
        ],
        out_specs=pl.BlockSpec((_PCHUNK, _W), lambda i: (i, 0)),
        out_shape=jax.ShapeDtypeStruct((_NUM_EMB, _W), jnp.float32),
    )(tableT, w1p, b1p)


def _sc_gather(h_table, idx):
    mesh = plsc.VectorSubcoreMesh(core_axis_name="c", subcore_axis_name="s")

    @functools.partial(
        pl.kernel,
        out_type=jax.ShapeDtypeStruct((_N_IDX, _W), jnp.float32),
        mesh=mesh,
        scratch_types=[
            pltpu.VMEM((_GCHUNK,), jnp.int32),
            pltpu.VMEM((_GCHUNK, _W), jnp.float32),
            pltpu.SemaphoreType.DMA,
        ],
    )
    def gather_kernel(table_hbm, idx_hbm, out_hbm, idx_v, rows_v, sem):
        wid = jax.lax.axis_index("s") * 2 + jax.lax.axis_index("c")
        base = wid * _B_PER_W

        @pl.loop(0, _N_GCH)
        def _(j):
            off = base + j * _GCHUNK
            pltpu.sync_copy(idx_hbm.at[pl.ds(off, _GCHUNK)], idx_v)
            pltpu.async_copy(table_hbm.at[idx_v], rows_v, sem).wait()
            pltpu.sync_copy(rows_v, out_hbm.at[pl.ds(off, _GCHUNK)])

    return gather_kernel(h_table, idx)


def _mlp_body(e_ref, w2a_ref, b2a_ref, w2b_ref, b2b_ref, w2c_ref, b2c_ref,
              o_ref):
    rows = e_ref.shape[1]
    acc = jnp.zeros((rows, _CLASS), jnp.float32)
    for s in range(_SEQ):
        es = e_ref[s].astype(jnp.bfloat16)
        acc = acc + jnp.dot(es, w2a_ref[s],
                            preferred_element_type=jnp.float32)
    h2 = jnp.tanh(acc + b2a_ref[...]).astype(jnp.bfloat16)
    h3 = jnp.tanh(
        jnp.dot(h2, w2b_ref[...], preferred_element_type=jnp.float32)
        + b2b_ref[...]).astype(jnp.bfloat16)
    o_ref[...] = (
        jnp.dot(h3, w2c_ref[...], preferred_element_type=jnp.float32)
        + b2c_ref[...])


def _tc_mlp(e3, w2ap, b2a, w2b, b2b, w2c, b2c):
    return pl.pallas_call(
        _mlp_body,
        grid=(_BATCH // _CHUNK,),
        in_specs=[
            pl.BlockSpec((_SEQ, _CHUNK, _W), lambda i: (0, i, 0)),
            pl.BlockSpec((_SEQ, _W, _CLASS), lambda i: (0, 0, 0)),
            pl.BlockSpec((1, _CLASS), lambda i: (0, 0)),
            pl.BlockSpec((_CLASS, 64), lambda i: (0, 0)),
            pl.BlockSpec((1, 64), lambda i: (0, 0)),
            pl.BlockSpec((64, _CLASS), lambda i: (0, 0)),
            pl.BlockSpec((1, _CLASS), lambda i: (0, 0)),
        ],
        out_specs=pl.BlockSpec((_CHUNK, _CLASS), lambda i: (i, 0)),
        out_shape=jax.ShapeDtypeStruct((_BATCH, _CLASS), jnp.float32),
    )(e3, w2ap, b2a, w2b, b2b, w2c, b2c)


def kernel(x, table, W1, b1, W2a, b2a, W2b, b2b, W2c, b2c):
    w1p = jnp.zeros((_EMB, _W), jnp.float32).at[:, :_HIDD].set(W1)
    w1p = w1p.astype(jnp.bfloat16)
    b1p = jnp.zeros((1, _W), jnp.float32).at[0, :_HIDD].set(b1)
    h_table = _tc_project(table.T, w1p, b1p)

    idx_t = x.T.reshape(_N_IDX)
    e = _sc_gather(h_table, idx_t)
    e3 = e.reshape(_SEQ, _BATCH, _W)

    w2ap = jnp.zeros((_SEQ, _W, _CLASS), jnp.float32)
    w2ap = w2ap.at[:, :_HIDD, :].set(W2a.reshape(_SEQ, _HIDD, _CLASS))
    w2ap = w2ap.astype(jnp.bfloat16)
    return _tc_mlp(e3, w2ap, b2a.reshape(1, -1), W2b.astype(jnp.bfloat16),
                   b2b.reshape(1, -1), W2c.astype(jnp.bfloat16),
                   b2c.reshape(1, -1))
